# baseline (speedup 1.0000x reference)
import functools
import jax
import jax.numpy as jnp
from jax import lax
from jax.experimental import pallas as pl
from jax.experimental.pallas import tpu as pltpu
from jax.experimental.pallas import tpu_sc as plsc

NC = 2
NS = 16
NW = NC * NS
CHUNK = 80


def _prep_body(hh_ref, w1a_ref, w1b_ref, p1_ref, p2_ref):
    hh = hh_ref[...]
    p1_ref[...] = jnp.dot(hh, w1a_ref[...], preferred_element_type=jnp.float32)
    p2_ref[...] = jnp.dot(hh, w1b_ref[...], preferred_element_type=jnp.float32)


def _gather128_body(e_per_w, n_iter,
                    p1_hbm, p2_hbm, src_hbm, dst_hbm,
                    gs_hbm, gd_hbm,
                    isrc, idst, bs, bd, sem0, sem1):
    wid = lax.axis_index("c") * NS + lax.axis_index("s")
    base = wid * e_per_w

    def step(i, _):
        off = base + i * CHUNK
        pltpu.sync_copy(src_hbm.at[pl.ds(off, CHUNK)], isrc)
        pltpu.sync_copy(dst_hbm.at[pl.ds(off, CHUNK)], idst)
        c0 = pltpu.async_copy(p1_hbm.at[isrc], bs, sem0)
        c1 = pltpu.async_copy(p2_hbm.at[idst], bd, sem1)
        c0.wait(); c1.wait()
        pltpu.sync_copy(bs, gs_hbm.at[pl.ds(off, CHUNK)])
        pltpu.sync_copy(bd, gd_hbm.at[pl.ds(off, CHUNK)])
        return 0

    lax.fori_loop(0, n_iter, step, 0)


def _gather16_body(e_per_w, n_iter,
                   xt_hbm, src_hbm, dst_hbm,
                   gxs_hbm, gxd_hbm,
                   isrc, idst, bxs, bxd, sem0, sem1):
    wid = lax.axis_index("c") * NS + lax.axis_index("s")
    base = wid * e_per_w

    def step(i, _):
        off = base + i * CHUNK
        pltpu.sync_copy(src_hbm.at[pl.ds(off, CHUNK)], isrc)
        pltpu.sync_copy(dst_hbm.at[pl.ds(off, CHUNK)], idst)
        c0 = pltpu.async_copy(xt_hbm.at[isrc], bxs, sem0)
        c1 = pltpu.async_copy(xt_hbm.at[idst], bxd, sem1)
        c0.wait(); c1.wait()
        pltpu.sync_copy(bxs, gxs_hbm.at[pl.ds(off, CHUNK)])
        pltpu.sync_copy(bxd, gxd_hbm.at[pl.ds(off, CHUNK)])
        return 0

    lax.fori_loop(0, n_iter, step, 0)


def _edge_body(gs_ref, gd_ref, gxs_ref, gxd_ref,
               wr_ref, be1_ref, w2_ref, be2_ref, wc1_ref, bc1_ref, wc2_ref,
               ef_ref, t16_ref):
    diff = gxs_ref[...] - gxd_ref[...]
    radial = jnp.sum(diff * diff, axis=1, keepdims=True)
    p = gs_ref[...] + gd_ref[...] + radial * wr_ref[...] + be1_ref[...]
    e1 = p * jax.nn.sigmoid(p)
    ef = jnp.dot(e1, w2_ref[...], preferred_element_type=jnp.float32) + be2_ref[...]
    ef = ef * jax.nn.sigmoid(ef)
    g = jnp.dot(ef, wc1_ref[...], preferred_element_type=jnp.float32) + bc1_ref[...]
    g = g * jax.nn.sigmoid(g)
    cm = jnp.sum(g * wc2_ref[...], axis=1, keepdims=True)
    trans = jnp.clip(diff * cm, -1000.0, 1000.0)
    lane = lax.broadcasted_iota(jnp.int32, trans.shape, 1)
    t16_ref[...] = jnp.where(lane == 3, 1.0, trans)
    ef_ref[...] = ef


def _scatter128_body(n_nodes, e_per_w, n_iter,
                     dst_hbm, ef_hbm, z128_hbm, o128_hbm,
                     sh128, idx, b128):
    c = lax.axis_index("c")
    s = lax.axis_index("s")
    wid = c * NS + s
    base = wid * e_per_w

    @pl.when(s == 0)
    def _init():
        pltpu.sync_copy(z128_hbm, sh128)

    plsc.subcore_barrier()

    def step(i, _):
        off = base + i * CHUNK
        pltpu.sync_copy(dst_hbm.at[pl.ds(off, CHUNK)], idx)
        pltpu.sync_copy(ef_hbm.at[pl.ds(off, CHUNK)], b128)
        pltpu.sync_copy(b128, sh128.at[idx], add=True)
        return 0

    lax.fori_loop(0, n_iter, step, 0)
    plsc.subcore_barrier()

    @pl.when(s == 0)
    def _flush():
        pltpu.sync_copy(sh128, o128_hbm.at[c])


def _scatter16_body(n_nodes, e_per_w, n_iter,
                    dst_hbm, t16_hbm, z16_hbm, o16_hbm,
                    sh16, idx, b16):
    c = lax.axis_index("c")
    s = lax.axis_index("s")
    wid = c * NS + s
    base = wid * e_per_w

    @pl.when(s == 0)
    def _init():
        pltpu.sync_copy(z16_hbm, sh16)

    plsc.subcore_barrier()

    def step(i, _):
        off = base + i * CHUNK
        pltpu.sync_copy(dst_hbm.at[pl.ds(off, CHUNK)], idx)
        pltpu.sync_copy(t16_hbm.at[pl.ds(off, CHUNK)], b16)
        pltpu.sync_copy(b16, sh16.at[idx], add=True)
        return 0

    lax.fori_loop(0, n_iter, step, 0)
    plsc.subcore_barrier()

    @pl.when(s == 0)
    def _flush():
        pltpu.sync_copy(sh16, o16_hbm.at[c])


def _node_body(hh_ref, x16_ref, s0a_ref, s1a_ref, s0b_ref, s1b_ref,
               wn1a_ref, wn1b_ref, bn1_ref, wn2_ref, bn2_ref,
               coord_ref, h_ref):
    hh = hh_ref[...]
    ef_sum = s0a_ref[...] + s1a_ref[...]
    t16 = s0b_ref[...] + s1b_ref[...]
    deg = t16[:, 3:4]
    deg_safe = jnp.maximum(deg, 1.0)
    x16 = x16_ref[...]
    xc = jnp.clip(x16, -1000.0, 1000.0)
    coord_ref[...] = jnp.where(deg > 0, xc + t16 / deg_safe, x16)
    a = (jnp.dot(hh, wn1a_ref[...], preferred_element_type=jnp.float32)
         + jnp.dot(ef_sum, wn1b_ref[...], preferred_element_type=jnp.float32)
         + bn1_ref[...])
    a = a * jax.nn.sigmoid(a)
    h = jnp.dot(a, wn2_ref[...], preferred_element_type=jnp.float32) + bn2_ref[...] + hh
    h_ref[...] = jnp.where(deg > 0, h, hh)


@jax.jit
def kernel(x, hh, edge_index, We1, be1, We2, be2, Wc1, bc1, Wc2, Wn1, bn1, Wn2, bn2):
    N, D = hh.shape
    E = edge_index.shape[1]
    H = We2.shape[0]
    f32 = jnp.float32
    src = edge_index[0]
    dst = edge_index[1]
    x16 = jnp.pad(x, ((0, 0), (0, 16 - x.shape[1])))

    e_per_w = E // NW
    n_iter = e_per_w // CHUNK

    p1, p2 = pl.pallas_call(
        _prep_body,
        out_shape=(jax.ShapeDtypeStruct((N, H), f32),
                   jax.ShapeDtypeStruct((N, H), f32)),
    )(hh, We1[1:1 + D], We1[1 + D:1 + 2 * D])

    gather128 = pl.kernel(
        functools.partial(_gather128_body, e_per_w, n_iter),
        out_type=(jax.ShapeDtypeStruct((E, H), f32),
                  jax.ShapeDtypeStruct((E, H), f32)),
        mesh=plsc.VectorSubcoreMesh(core_axis_name="c", subcore_axis_name="s"),
        compiler_params=pltpu.CompilerParams(use_tc_tiling_on_sc=True),
        scratch_types=(
            pltpu.VMEM((CHUNK,), jnp.int32),
            pltpu.VMEM((CHUNK,), jnp.int32),
            pltpu.VMEM((CHUNK, H), f32),
            pltpu.VMEM((CHUNK, H), f32),
            pltpu.SemaphoreType.DMA,
            pltpu.SemaphoreType.DMA,
        ),
    )
    gs, gd = gather128(p1, p2, src, dst)

    gather16 = pl.kernel(
        functools.partial(_gather16_body, e_per_w, n_iter),
        out_type=(jax.ShapeDtypeStruct((E, 16), f32),
                  jax.ShapeDtypeStruct((E, 16), f32)),
        mesh=plsc.VectorSubcoreMesh(core_axis_name="c", subcore_axis_name="s"),
        compiler_params=pltpu.CompilerParams(use_tc_tiling_on_sc=False),
        scratch_types=(
            pltpu.VMEM((CHUNK,), jnp.int32),
            pltpu.VMEM((CHUNK,), jnp.int32),
            pltpu.VMEM((CHUNK, 16), f32),
            pltpu.VMEM((CHUNK, 16), f32),
            pltpu.SemaphoreType.DMA,
            pltpu.SemaphoreType.DMA,
        ),
    )
    gxs, gxd = gather16(x16, src, dst)

    B = 1280
    grid = E // B
    ef, t16 = pl.pallas_call(
        _edge_body,
        grid=(grid,),
        in_specs=[
            pl.BlockSpec((B, H), lambda i: (i, 0)),
            pl.BlockSpec((B, H), lambda i: (i, 0)),
            pl.BlockSpec((B, 16), lambda i: (i, 0)),
            pl.BlockSpec((B, 16), lambda i: (i, 0)),
            pl.BlockSpec((1, H), lambda i: (0, 0)),
            pl.BlockSpec((1, H), lambda i: (0, 0)),
            pl.BlockSpec((H, H), lambda i: (0, 0)),
            pl.BlockSpec((1, H), lambda i: (0, 0)),
            pl.BlockSpec((H, H), lambda i: (0, 0)),
            pl.BlockSpec((1, H), lambda i: (0, 0)),
            pl.BlockSpec((1, H), lambda i: (0, 0)),
        ],
        out_specs=[
            pl.BlockSpec((B, H), lambda i: (i, 0)),
            pl.BlockSpec((B, 16), lambda i: (i, 0)),
        ],
        out_shape=(jax.ShapeDtypeStruct((E, H), f32),
                   jax.ShapeDtypeStruct((E, 16), f32)),
    )(gs, gd, gxs, gxd,
      We1[0:1], be1.reshape(1, H), We2, be2.reshape(1, H),
      Wc1, bc1.reshape(1, H), Wc2.reshape(1, H))

    scatter128 = pl.kernel(
        functools.partial(_scatter128_body, N, e_per_w, n_iter),
        out_type=jax.ShapeDtypeStruct((NC, N, H), f32),
        mesh=plsc.VectorSubcoreMesh(core_axis_name="c", subcore_axis_name="s"),
        compiler_params=pltpu.CompilerParams(use_tc_tiling_on_sc=True),
        scratch_types=(
            pltpu.VMEM_SHARED((N, H), f32),
            pltpu.VMEM((CHUNK,), jnp.int32),
            pltpu.VMEM((CHUNK, H), f32),
        ),
    )
    o128 = scatter128(dst, ef, jnp.zeros((N, H), f32))

    scatter16 = pl.kernel(
        functools.partial(_scatter16_body, N, e_per_w, n_iter),
        out_type=jax.ShapeDtypeStruct((NC, N, 16), f32),
        mesh=plsc.VectorSubcoreMesh(core_axis_name="c", subcore_axis_name="s"),
        compiler_params=pltpu.CompilerParams(use_tc_tiling_on_sc=False),
        scratch_types=(
            pltpu.VMEM_SHARED((N, 16), f32),
            pltpu.VMEM((CHUNK,), jnp.int32),
            pltpu.VMEM((CHUNK, 16), f32),
        ),
    )
    o16 = scatter16(dst, t16, jnp.zeros((N, 16), f32))

    coord16, h_out = pl.pallas_call(
        _node_body,
        out_shape=(jax.ShapeDtypeStruct((N, 16), f32),
                   jax.ShapeDtypeStruct((N, D), f32)),
    )(hh, x16, o128[0], o128[1], o16[0], o16[1],
      Wn1[:D], Wn1[D:], bn1.reshape(1, H), Wn2, bn2.reshape(1, D))

    return coord16[:, :3], h_out


# speedup vs baseline: 1.0320x; 1.0320x over previous
import functools
import jax
import jax.numpy as jnp
from jax import lax
from jax.experimental import pallas as pl
from jax.experimental.pallas import tpu as pltpu
from jax.experimental.pallas import tpu_sc as plsc

NC = 2
NS = 16
NW = NC * NS
CHUNK = 80


def _prep_body(hh_ref, w1a_ref, w1b_ref, p1_ref, p2_ref):
    hh = hh_ref[...]
    p1_ref[...] = jnp.dot(hh, w1a_ref[...], preferred_element_type=jnp.float32)
    p2_ref[...] = jnp.dot(hh, w1b_ref[...], preferred_element_type=jnp.float32)


def _gather128_body(e_per_w, n_iter,
                    p1_hbm, p2_hbm, src_hbm, dst_hbm,
                    gs_hbm, gd_hbm,
                    isrc, idst, bs, bd, sem0, sem1):
    wid = lax.axis_index("c") * NS + lax.axis_index("s")
    base = wid * e_per_w

    def step(i, _):
        off = base + i * CHUNK
        pltpu.sync_copy(src_hbm.at[pl.ds(off, CHUNK)], isrc)
        pltpu.sync_copy(dst_hbm.at[pl.ds(off, CHUNK)], idst)
        c0 = pltpu.async_copy(p1_hbm.at[isrc], bs, sem0)
        c1 = pltpu.async_copy(p2_hbm.at[idst], bd, sem1)
        c0.wait(); c1.wait()
        pltpu.sync_copy(bs, gs_hbm.at[pl.ds(off, CHUNK)])
        pltpu.sync_copy(bd, gd_hbm.at[pl.ds(off, CHUNK)])
        return 0

    lax.fori_loop(0, n_iter, step, 0)


def _gather16_body(e_per_w, n_iter,
                   xt_hbm, src_hbm, dst_hbm,
                   diff_hbm,
                   isrc, idst, bxs, bxd, bdf, sem0, sem1):
    wid = lax.axis_index("c") * NS + lax.axis_index("s")
    base = wid * e_per_w

    def step(i, _):
        off = base + i * CHUNK
        pltpu.sync_copy(src_hbm.at[pl.ds(off, CHUNK)], isrc)
        pltpu.sync_copy(dst_hbm.at[pl.ds(off, CHUNK)], idst)
        c0 = pltpu.async_copy(xt_hbm.at[isrc], bxs, sem0)
        c1 = pltpu.async_copy(xt_hbm.at[idst], bxd, sem1)
        c0.wait(); c1.wait()
        bdf[...] = bxs[...] - bxd[...]
        pltpu.sync_copy(bdf, diff_hbm.at[pl.ds(off, CHUNK)])
        return 0

    lax.fori_loop(0, n_iter, step, 0)


def _edge_body(gs_ref, gd_ref, diff_ref,
               wr_ref, be1_ref, w2_ref, be2_ref, wc1_ref, bc1_ref, wc2_ref,
               ef_ref, t16_ref):
    d = diff_ref[...]
    rad = jnp.sum(d * d, axis=1, keepdims=True)
    wr = wr_ref[...]
    prad = rad * wr
    p = gs_ref[...] + gd_ref[...] + prad + be1_ref[...]
    e1 = p * jax.nn.sigmoid(p)
    ef = jnp.dot(e1, w2_ref[...], preferred_element_type=jnp.float32) + be2_ref[...]
    ef = ef * jax.nn.sigmoid(ef)
    g = jnp.dot(ef, wc1_ref[...], preferred_element_type=jnp.float32) + bc1_ref[...]
    g = g * jax.nn.sigmoid(g)
    cm = jnp.sum(g * wc2_ref[...], axis=1, keepdims=True)
    t = jnp.clip(d * cm, -1000.0, 1000.0)
    lane = lax.broadcasted_iota(jnp.int32, t.shape, 1)
    t16_ref[...] = jnp.where(lane == 3, 1.0, t)
    ef_ref[...] = ef


def _scatter128_body(n_nodes, e_per_w, n_iter,
                     dst_hbm, ef_hbm, z128_hbm, o128_hbm,
                     sh128, idx, b128):
    c = lax.axis_index("c")
    s = lax.axis_index("s")
    wid = c * NS + s
    base = wid * e_per_w

    @pl.when(s == 0)
    def _init():
        pltpu.sync_copy(z128_hbm, sh128)

    plsc.subcore_barrier()

    def step(i, _):
        off = base + i * CHUNK
        pltpu.sync_copy(dst_hbm.at[pl.ds(off, CHUNK)], idx)
        pltpu.sync_copy(ef_hbm.at[pl.ds(off, CHUNK)], b128)
        pltpu.sync_copy(b128, sh128.at[idx], add=True)
        return 0

    lax.fori_loop(0, n_iter, step, 0)
    plsc.subcore_barrier()

    @pl.when(s == 0)
    def _flush():
        pltpu.sync_copy(sh128, o128_hbm.at[c])


def _scatter16_body(n_nodes, e_per_w, n_iter,
                    dst_hbm, t16_hbm, z16_hbm, o16_hbm,
                    sh16, idx, b16):
    c = lax.axis_index("c")
    s = lax.axis_index("s")
    wid = c * NS + s
    base = wid * e_per_w

    @pl.when(s == 0)
    def _init():
        pltpu.sync_copy(z16_hbm, sh16)

    plsc.subcore_barrier()

    def step(i, _):
        off = base + i * CHUNK
        pltpu.sync_copy(dst_hbm.at[pl.ds(off, CHUNK)], idx)
        pltpu.sync_copy(t16_hbm.at[pl.ds(off, CHUNK)], b16)
        pltpu.sync_copy(b16, sh16.at[idx], add=True)
        return 0

    lax.fori_loop(0, n_iter, step, 0)
    plsc.subcore_barrier()

    @pl.when(s == 0)
    def _flush():
        pltpu.sync_copy(sh16, o16_hbm.at[c])


def _node_body(hh_ref, x16_ref, s0a_ref, s1a_ref, s0b_ref, s1b_ref,
               wn1a_ref, wn1b_ref, bn1_ref, wn2_ref, bn2_ref,
               coord_ref, h_ref):
    hh = hh_ref[...]
    ef_sum = s0a_ref[...] + s1a_ref[...]
    t16 = s0b_ref[...] + s1b_ref[...]
    deg = t16[:, 3:4]
    deg_safe = jnp.maximum(deg, 1.0)
    x16 = x16_ref[...]
    xc = jnp.clip(x16, -1000.0, 1000.0)
    coord_ref[...] = jnp.where(deg > 0, xc + t16 / deg_safe, x16)
    a = (jnp.dot(hh, wn1a_ref[...], preferred_element_type=jnp.float32)
         + jnp.dot(ef_sum, wn1b_ref[...], preferred_element_type=jnp.float32)
         + bn1_ref[...])
    a = a * jax.nn.sigmoid(a)
    h = jnp.dot(a, wn2_ref[...], preferred_element_type=jnp.float32) + bn2_ref[...] + hh
    h_ref[...] = jnp.where(deg > 0, h, hh)


@jax.jit
def kernel(x, hh, edge_index, We1, be1, We2, be2, Wc1, bc1, Wc2, Wn1, bn1, Wn2, bn2):
    N, D = hh.shape
    E = edge_index.shape[1]
    H = We2.shape[0]
    f32 = jnp.float32
    src = edge_index[0]
    dst = edge_index[1]
    x16 = jnp.pad(x, ((0, 0), (0, 16 - x.shape[1])))

    e_per_w = E // NW
    n_iter = e_per_w // CHUNK

    p1, p2 = pl.pallas_call(
        _prep_body,
        out_shape=(jax.ShapeDtypeStruct((N, H), f32),
                   jax.ShapeDtypeStruct((N, H), f32)),
    )(hh, We1[1:1 + D], We1[1 + D:1 + 2 * D])

    gather128 = pl.kernel(
        functools.partial(_gather128_body, e_per_w, n_iter),
        out_type=(jax.ShapeDtypeStruct((E, H), f32),
                  jax.ShapeDtypeStruct((E, H), f32)),
        mesh=plsc.VectorSubcoreMesh(core_axis_name="c", subcore_axis_name="s"),
        compiler_params=pltpu.CompilerParams(use_tc_tiling_on_sc=True),
        scratch_types=(
            pltpu.VMEM((CHUNK,), jnp.int32),
            pltpu.VMEM((CHUNK,), jnp.int32),
            pltpu.VMEM((CHUNK, H), f32),
            pltpu.VMEM((CHUNK, H), f32),
            pltpu.SemaphoreType.DMA,
            pltpu.SemaphoreType.DMA,
        ),
    )
    gs, gd = gather128(p1, p2, src, dst)

    gather16 = pl.kernel(
        functools.partial(_gather16_body, e_per_w, n_iter),
        out_type=jax.ShapeDtypeStruct((E, 16), f32),
        mesh=plsc.VectorSubcoreMesh(core_axis_name="c", subcore_axis_name="s"),
        compiler_params=pltpu.CompilerParams(use_tc_tiling_on_sc=False),
        scratch_types=(
            pltpu.VMEM((CHUNK,), jnp.int32),
            pltpu.VMEM((CHUNK,), jnp.int32),
            pltpu.VMEM((CHUNK, 16), f32),
            pltpu.VMEM((CHUNK, 16), f32),
            pltpu.VMEM((CHUNK, 16), f32),
            pltpu.SemaphoreType.DMA,
            pltpu.SemaphoreType.DMA,
        ),
    )
    diff = gather16(x16, src, dst)

    B = 1280
    grid = E // B
    ef, t16 = pl.pallas_call(
        _edge_body,
        grid=(grid,),
        in_specs=[
            pl.BlockSpec((B, H), lambda i: (i, 0)),
            pl.BlockSpec((B, H), lambda i: (i, 0)),
            pl.BlockSpec((B, 16), lambda i: (i, 0)),
            pl.BlockSpec((1, H), lambda i: (0, 0)),
            pl.BlockSpec((1, H), lambda i: (0, 0)),
            pl.BlockSpec((H, H), lambda i: (0, 0)),
            pl.BlockSpec((1, H), lambda i: (0, 0)),
            pl.BlockSpec((H, H), lambda i: (0, 0)),
            pl.BlockSpec((1, H), lambda i: (0, 0)),
            pl.BlockSpec((1, H), lambda i: (0, 0)),
        ],
        out_specs=[
            pl.BlockSpec((B, H), lambda i: (i, 0)),
            pl.BlockSpec((B, 16), lambda i: (i, 0)),
        ],
        out_shape=(jax.ShapeDtypeStruct((E, H), f32),
                   jax.ShapeDtypeStruct((E, 16), f32)),
    )(gs, gd, diff,
      We1[0:1], be1.reshape(1, H), We2, be2.reshape(1, H),
      Wc1, bc1.reshape(1, H), Wc2.reshape(1, H))

    scatter128 = pl.kernel(
        functools.partial(_scatter128_body, N, e_per_w, n_iter),
        out_type=jax.ShapeDtypeStruct((NC, N, H), f32),
        mesh=plsc.VectorSubcoreMesh(core_axis_name="c", subcore_axis_name="s"),
        compiler_params=pltpu.CompilerParams(use_tc_tiling_on_sc=True),
        scratch_types=(
            pltpu.VMEM_SHARED((N, H), f32),
            pltpu.VMEM((CHUNK,), jnp.int32),
            pltpu.VMEM((CHUNK, H), f32),
        ),
    )
    o128 = scatter128(dst, ef, jnp.zeros((N, H), f32))

    scatter16 = pl.kernel(
        functools.partial(_scatter16_body, N, e_per_w, n_iter),
        out_type=jax.ShapeDtypeStruct((NC, N, 16), f32),
        mesh=plsc.VectorSubcoreMesh(core_axis_name="c", subcore_axis_name="s"),
        compiler_params=pltpu.CompilerParams(use_tc_tiling_on_sc=False),
        scratch_types=(
            pltpu.VMEM_SHARED((N, 16), f32),
            pltpu.VMEM((CHUNK,), jnp.int32),
            pltpu.VMEM((CHUNK, 16), f32),
        ),
    )
    o16 = scatter16(dst, t16, jnp.zeros((N, 16), f32))

    coord16, h_out = pl.pallas_call(
        _node_body,
        out_shape=(jax.ShapeDtypeStruct((N, 16), f32),
                   jax.ShapeDtypeStruct((N, D), f32)),
    )(hh, x16, o128[0], o128[1], o16[0], o16[1],
      Wn1[:D], Wn1[D:], bn1.reshape(1, H), Wn2, bn2.reshape(1, D))

    return coord16[:, :3], h_out


# speedup vs baseline: 1.4030x; 1.3594x over previous
import functools
import jax
import jax.numpy as jnp
from jax import lax
from jax.experimental import pallas as pl
from jax.experimental.pallas import tpu as pltpu
from jax.experimental.pallas import tpu_sc as plsc

NC = 2
NS = 16
NW = NC * NS
CHUNK = 80


def _prep_body(hh_ref, w1a_ref, w1b_ref, p1_ref, p2_ref):
    hh = hh_ref[...]
    p1_ref[...] = jnp.dot(hh, w1a_ref[...], preferred_element_type=jnp.float32)
    p2_ref[...] = jnp.dot(hh, w1b_ref[...], preferred_element_type=jnp.float32)


def _gather128_body(n_iter,
                    p1_hbm, p2_hbm, src3_hbm, dst3_hbm,
                    gs_hbm, gd_hbm,
                    isrc, idst, bsA, bdA, bsB, bdB,
                    sA0, sA1, sB0, sB1):
    wid = lax.axis_index("c") * NS + lax.axis_index("s")
    base = wid * (n_iter * CHUNK)

    pltpu.sync_copy(src3_hbm.at[wid], isrc)
    pltpu.sync_copy(dst3_hbm.at[wid], idst)

    pltpu.async_copy(p1_hbm.at[isrc.at[0]], bsA, sA0)
    pltpu.async_copy(p2_hbm.at[idst.at[0]], bdA, sA1)

    def pair(j, _):
        i0 = 2 * j
        i1 = i0 + 1
        i2 = i0 + 2
        pltpu.make_async_copy(p1_hbm.at[isrc.at[i0]], bsA, sA0).wait()
        pltpu.make_async_copy(p2_hbm.at[idst.at[i0]], bdA, sA1).wait()
        pltpu.async_copy(p1_hbm.at[isrc.at[i1]], bsB, sB0)
        pltpu.async_copy(p2_hbm.at[idst.at[i1]], bdB, sB1)
        pltpu.sync_copy(bsA, gs_hbm.at[pl.ds(base + i0 * CHUNK, CHUNK)])
        pltpu.sync_copy(bdA, gd_hbm.at[pl.ds(base + i0 * CHUNK, CHUNK)])
        pltpu.make_async_copy(p1_hbm.at[isrc.at[i1]], bsB, sB0).wait()
        pltpu.make_async_copy(p2_hbm.at[idst.at[i1]], bdB, sB1).wait()
        pltpu.async_copy(p1_hbm.at[isrc.at[i2]], bsA, sA0)
        pltpu.async_copy(p2_hbm.at[idst.at[i2]], bdA, sA1)
        pltpu.sync_copy(bsB, gs_hbm.at[pl.ds(base + i1 * CHUNK, CHUNK)])
        pltpu.sync_copy(bdB, gd_hbm.at[pl.ds(base + i1 * CHUNK, CHUNK)])
        return 0

    lax.fori_loop(0, (n_iter - 1) // 2, pair, 0)

    last = n_iter - 1
    pltpu.make_async_copy(p1_hbm.at[isrc.at[last]], bsA, sA0).wait()
    pltpu.make_async_copy(p2_hbm.at[idst.at[last]], bdA, sA1).wait()
    pltpu.sync_copy(bsA, gs_hbm.at[pl.ds(base + last * CHUNK, CHUNK)])
    pltpu.sync_copy(bdA, gd_hbm.at[pl.ds(base + last * CHUNK, CHUNK)])


def _gather16_body(n_iter,
                   xt_hbm, src3_hbm, dst3_hbm,
                   diff_hbm,
                   isrc, idst, bxsA, bxdA, bxsB, bxdB, bdfA, bdfB,
                   sA0, sA1, sB0, sB1):
    wid = lax.axis_index("c") * NS + lax.axis_index("s")
    base = wid * (n_iter * CHUNK)

    pltpu.sync_copy(src3_hbm.at[wid], isrc)
    pltpu.sync_copy(dst3_hbm.at[wid], idst)

    pltpu.async_copy(xt_hbm.at[isrc.at[0]], bxsA, sA0)
    pltpu.async_copy(xt_hbm.at[idst.at[0]], bxdA, sA1)

    def pair(j, _):
        i0 = 2 * j
        i1 = i0 + 1
        i2 = i0 + 2
        pltpu.make_async_copy(xt_hbm.at[isrc.at[i0]], bxsA, sA0).wait()
        pltpu.make_async_copy(xt_hbm.at[idst.at[i0]], bxdA, sA1).wait()
        pltpu.async_copy(xt_hbm.at[isrc.at[i1]], bxsB, sB0)
        pltpu.async_copy(xt_hbm.at[idst.at[i1]], bxdB, sB1)
        bdfA[...] = bxsA[...] - bxdA[...]
        pltpu.sync_copy(bdfA, diff_hbm.at[pl.ds(base + i0 * CHUNK, CHUNK)])
        pltpu.make_async_copy(xt_hbm.at[isrc.at[i1]], bxsB, sB0).wait()
        pltpu.make_async_copy(xt_hbm.at[idst.at[i1]], bxdB, sB1).wait()
        pltpu.async_copy(xt_hbm.at[isrc.at[i2]], bxsA, sA0)
        pltpu.async_copy(xt_hbm.at[idst.at[i2]], bxdA, sA1)
        bdfB[...] = bxsB[...] - bxdB[...]
        pltpu.sync_copy(bdfB, diff_hbm.at[pl.ds(base + i1 * CHUNK, CHUNK)])
        return 0

    lax.fori_loop(0, (n_iter - 1) // 2, pair, 0)

    last = n_iter - 1
    pltpu.make_async_copy(xt_hbm.at[isrc.at[last]], bxsA, sA0).wait()
    pltpu.make_async_copy(xt_hbm.at[idst.at[last]], bxdA, sA1).wait()
    bdfA[...] = bxsA[...] - bxdA[...]
    pltpu.sync_copy(bdfA, diff_hbm.at[pl.ds(base + last * CHUNK, CHUNK)])


def _edge_body(gs_ref, gd_ref, diff_ref,
               wr_ref, be1_ref, w2_ref, be2_ref, wc1_ref, bc1_ref, wc2_ref,
               ef_ref, t16_ref):
    d = diff_ref[...]
    rad = jnp.sum(d * d, axis=1, keepdims=True)
    wr = wr_ref[...]
    prad = rad * wr
    p = gs_ref[...] + gd_ref[...] + prad + be1_ref[...]
    e1 = p * jax.nn.sigmoid(p)
    ef = jnp.dot(e1, w2_ref[...], preferred_element_type=jnp.float32) + be2_ref[...]
    ef = ef * jax.nn.sigmoid(ef)
    g = jnp.dot(ef, wc1_ref[...], preferred_element_type=jnp.float32) + bc1_ref[...]
    g = g * jax.nn.sigmoid(g)
    cm = jnp.sum(g * wc2_ref[...], axis=1, keepdims=True)
    t = jnp.clip(d * cm, -1000.0, 1000.0)
    lane = lax.broadcasted_iota(jnp.int32, t.shape, 1)
    t16_ref[...] = jnp.where(lane == 3, 1.0, t)
    ef_ref[...] = ef


def _scatter128_body(n_iter,
                     dst3_hbm, ef_hbm, z128_hbm, o128_hbm,
                     sh128, idx2, bA, bB, lA, lB):
    c = lax.axis_index("c")
    s = lax.axis_index("s")
    wid = c * NS + s
    base = wid * (n_iter * CHUNK)

    @pl.when(s == 0)
    def _init():
        pltpu.sync_copy(z128_hbm, sh128)

    pltpu.sync_copy(dst3_hbm.at[wid], idx2)
    pltpu.async_copy(ef_hbm.at[pl.ds(base, CHUNK)], bA, lA)

    plsc.subcore_barrier()

    def pair(j, _):
        i0 = 2 * j
        i1 = i0 + 1
        i2 = i0 + 2
        pltpu.make_async_copy(ef_hbm.at[pl.ds(base + i0 * CHUNK, CHUNK)], bA, lA).wait()
        pltpu.async_copy(ef_hbm.at[pl.ds(base + i1 * CHUNK, CHUNK)], bB, lB)
        pltpu.sync_copy(bA, sh128.at[idx2.at[i0]], add=True)
        pltpu.make_async_copy(ef_hbm.at[pl.ds(base + i1 * CHUNK, CHUNK)], bB, lB).wait()
        pltpu.async_copy(ef_hbm.at[pl.ds(base + i2 * CHUNK, CHUNK)], bA, lA)
        pltpu.sync_copy(bB, sh128.at[idx2.at[i1]], add=True)
        return 0

    lax.fori_loop(0, (n_iter - 1) // 2, pair, 0)

    last = n_iter - 1
    pltpu.make_async_copy(ef_hbm.at[pl.ds(base + last * CHUNK, CHUNK)], bA, lA).wait()
    pltpu.sync_copy(bA, sh128.at[idx2.at[last]], add=True)

    plsc.subcore_barrier()

    @pl.when(s == 0)
    def _flush():
        pltpu.sync_copy(sh128, o128_hbm.at[c])


def _scatter16_body(n_iter,
                    dst3_hbm, t16_hbm, z16_hbm, o16_hbm,
                    sh16, idx2, bA, bB, lA, lB):
    c = lax.axis_index("c")
    s = lax.axis_index("s")
    wid = c * NS + s
    base = wid * (n_iter * CHUNK)

    @pl.when(s == 0)
    def _init():
        pltpu.sync_copy(z16_hbm, sh16)

    pltpu.sync_copy(dst3_hbm.at[wid], idx2)
    pltpu.async_copy(t16_hbm.at[pl.ds(base, CHUNK)], bA, lA)

    plsc.subcore_barrier()

    def pair(j, _):
        i0 = 2 * j
        i1 = i0 + 1
        i2 = i0 + 2
        pltpu.make_async_copy(t16_hbm.at[pl.ds(base + i0 * CHUNK, CHUNK)], bA, lA).wait()
        pltpu.async_copy(t16_hbm.at[pl.ds(base + i1 * CHUNK, CHUNK)], bB, lB)
        pltpu.sync_copy(bA, sh16.at[idx2.at[i0]], add=True)
        pltpu.make_async_copy(t16_hbm.at[pl.ds(base + i1 * CHUNK, CHUNK)], bB, lB).wait()
        pltpu.async_copy(t16_hbm.at[pl.ds(base + i2 * CHUNK, CHUNK)], bA, lA)
        pltpu.sync_copy(bB, sh16.at[idx2.at[i1]], add=True)
        return 0

    lax.fori_loop(0, (n_iter - 1) // 2, pair, 0)

    last = n_iter - 1
    pltpu.make_async_copy(t16_hbm.at[pl.ds(base + last * CHUNK, CHUNK)], bA, lA).wait()
    pltpu.sync_copy(bA, sh16.at[idx2.at[last]], add=True)

    plsc.subcore_barrier()

    @pl.when(s == 0)
    def _flush():
        pltpu.sync_copy(sh16, o16_hbm.at[c])


def _node_body(hh_ref, x16_ref, s0a_ref, s1a_ref, s0b_ref, s1b_ref,
               wn1a_ref, wn1b_ref, bn1_ref, wn2_ref, bn2_ref,
               coord_ref, h_ref):
    hh = hh_ref[...]
    ef_sum = s0a_ref[...] + s1a_ref[...]
    t16 = s0b_ref[...] + s1b_ref[...]
    deg = t16[:, 3:4]
    deg_safe = jnp.maximum(deg, 1.0)
    x16 = x16_ref[...]
    xc = jnp.clip(x16, -1000.0, 1000.0)
    coord_ref[...] = jnp.where(deg > 0, xc + t16 / deg_safe, x16)
    a = (jnp.dot(hh, wn1a_ref[...], preferred_element_type=jnp.float32)
         + jnp.dot(ef_sum, wn1b_ref[...], preferred_element_type=jnp.float32)
         + bn1_ref[...])
    a = a * jax.nn.sigmoid(a)
    h = jnp.dot(a, wn2_ref[...], preferred_element_type=jnp.float32) + bn2_ref[...] + hh
    h_ref[...] = jnp.where(deg > 0, h, hh)


@jax.jit
def kernel(x, hh, edge_index, We1, be1, We2, be2, Wc1, bc1, Wc2, Wn1, bn1, Wn2, bn2):
    N, D = hh.shape
    E = edge_index.shape[1]
    H = We2.shape[0]
    f32 = jnp.float32
    src = edge_index[0]
    dst = edge_index[1]
    x16 = jnp.pad(x, ((0, 0), (0, 16 - x.shape[1])))

    e_per_w = E // NW
    n_iter = e_per_w // CHUNK
    src3 = src.reshape(NW, n_iter, CHUNK)
    dst3 = dst.reshape(NW, n_iter, CHUNK)

    p1, p2 = pl.pallas_call(
        _prep_body,
        out_shape=(jax.ShapeDtypeStruct((N, H), f32),
                   jax.ShapeDtypeStruct((N, H), f32)),
    )(hh, We1[1:1 + D], We1[1 + D:1 + 2 * D])

    gather128 = pl.kernel(
        functools.partial(_gather128_body, n_iter),
        out_type=(jax.ShapeDtypeStruct((E, H), f32),
                  jax.ShapeDtypeStruct((E, H), f32)),
        mesh=plsc.VectorSubcoreMesh(core_axis_name="c", subcore_axis_name="s"),
        compiler_params=pltpu.CompilerParams(use_tc_tiling_on_sc=True),
        scratch_types=(
            pltpu.VMEM((n_iter, CHUNK), jnp.int32),
            pltpu.VMEM((n_iter, CHUNK), jnp.int32),
            pltpu.VMEM((CHUNK, H), f32),
            pltpu.VMEM((CHUNK, H), f32),
            pltpu.VMEM((CHUNK, H), f32),
            pltpu.VMEM((CHUNK, H), f32),
            pltpu.SemaphoreType.DMA,
            pltpu.SemaphoreType.DMA,
            pltpu.SemaphoreType.DMA,
            pltpu.SemaphoreType.DMA,
        ),
    )
    gs, gd = gather128(p1, p2, src3, dst3)

    gather16 = pl.kernel(
        functools.partial(_gather16_body, n_iter),
        out_type=jax.ShapeDtypeStruct((E, 16), f32),
        mesh=plsc.VectorSubcoreMesh(core_axis_name="c", subcore_axis_name="s"),
        compiler_params=pltpu.CompilerParams(use_tc_tiling_on_sc=False),
        scratch_types=(
            pltpu.VMEM((n_iter, CHUNK), jnp.int32),
            pltpu.VMEM((n_iter, CHUNK), jnp.int32),
            pltpu.VMEM((CHUNK, 16), f32),
            pltpu.VMEM((CHUNK, 16), f32),
            pltpu.VMEM((CHUNK, 16), f32),
            pltpu.VMEM((CHUNK, 16), f32),
            pltpu.VMEM((CHUNK, 16), f32),
            pltpu.VMEM((CHUNK, 16), f32),
            pltpu.SemaphoreType.DMA,
            pltpu.SemaphoreType.DMA,
            pltpu.SemaphoreType.DMA,
            pltpu.SemaphoreType.DMA,
        ),
    )
    diff = gather16(x16, src3, dst3)

    B = 1280
    grid = E // B
    ef, t16 = pl.pallas_call(
        _edge_body,
        grid=(grid,),
        in_specs=[
            pl.BlockSpec((B, H), lambda i: (i, 0)),
            pl.BlockSpec((B, H), lambda i: (i, 0)),
            pl.BlockSpec((B, 16), lambda i: (i, 0)),
            pl.BlockSpec((1, H), lambda i: (0, 0)),
            pl.BlockSpec((1, H), lambda i: (0, 0)),
            pl.BlockSpec((H, H), lambda i: (0, 0)),
            pl.BlockSpec((1, H), lambda i: (0, 0)),
            pl.BlockSpec((H, H), lambda i: (0, 0)),
            pl.BlockSpec((1, H), lambda i: (0, 0)),
            pl.BlockSpec((1, H), lambda i: (0, 0)),
        ],
        out_specs=[
            pl.BlockSpec((B, H), lambda i: (i, 0)),
            pl.BlockSpec((B, 16), lambda i: (i, 0)),
        ],
        out_shape=(jax.ShapeDtypeStruct((E, H), f32),
                   jax.ShapeDtypeStruct((E, 16), f32)),
    )(gs, gd, diff,
      We1[0:1], be1.reshape(1, H), We2, be2.reshape(1, H),
      Wc1, bc1.reshape(1, H), Wc2.reshape(1, H))

    scatter128 = pl.kernel(
        functools.partial(_scatter128_body, n_iter),
        out_type=jax.ShapeDtypeStruct((NC, N, H), f32),
        mesh=plsc.VectorSubcoreMesh(core_axis_name="c", subcore_axis_name="s"),
        compiler_params=pltpu.CompilerParams(use_tc_tiling_on_sc=True),
        scratch_types=(
            pltpu.VMEM_SHARED((N, H), f32),
            pltpu.VMEM((n_iter, CHUNK), jnp.int32),
            pltpu.VMEM((CHUNK, H), f32),
            pltpu.VMEM((CHUNK, H), f32),
            pltpu.SemaphoreType.DMA,
            pltpu.SemaphoreType.DMA,
        ),
    )
    o128 = scatter128(dst3, ef, jnp.zeros((N, H), f32))

    scatter16 = pl.kernel(
        functools.partial(_scatter16_body, n_iter),
        out_type=jax.ShapeDtypeStruct((NC, N, 16), f32),
        mesh=plsc.VectorSubcoreMesh(core_axis_name="c", subcore_axis_name="s"),
        compiler_params=pltpu.CompilerParams(use_tc_tiling_on_sc=False),
        scratch_types=(
            pltpu.VMEM_SHARED((N, 16), f32),
            pltpu.VMEM((n_iter, CHUNK), jnp.int32),
            pltpu.VMEM((CHUNK, 16), f32),
            pltpu.VMEM((CHUNK, 16), f32),
            pltpu.SemaphoreType.DMA,
            pltpu.SemaphoreType.DMA,
        ),
    )
    o16 = scatter16(dst3, t16, jnp.zeros((N, 16), f32))

    coord16, h_out = pl.pallas_call(
        _node_body,
        out_shape=(jax.ShapeDtypeStruct((N, 16), f32),
                   jax.ShapeDtypeStruct((N, D), f32)),
    )(hh, x16, o128[0], o128[1], o16[0], o16[1],
      Wn1[:D], Wn1[D:], bn1.reshape(1, H), Wn2, bn2.reshape(1, D))

    return coord16[:, :3], h_out
